# Initial kernel scaffold; baseline (speedup 1.0000x reference)
import jax
import jax.numpy as jnp
from jax.experimental import pallas as pl


def kernel(input_bat, node_feature, type_feature, length_feature, lane_feature, adj_index, adj_values, node_tab, type_tab, length_tab, lane_tab, gcn_W, gcn_b, W_ih, W_hh, b_ih, b_hh, lin_W, lin_b):
    raise NotImplementedError("write your pallas kernel here")


# speedup vs baseline: 3.5887x; 3.5887x over previous
import functools

import jax
import jax.numpy as jnp
from jax import lax
from jax.experimental import pallas as pl
from jax.experimental.pallas import tpu as pltpu
from jax.experimental.pallas import tpu_sc as plsc

_N = 10000
_E = 320000
_HID = 128
_B = 64
_L = 50
_BL = _B * _L

_NC = 2
_NS = 16
_NW = _NC * _NS
_EPW = _E // _NW
_C = 80
_NCHUNK = _EPW // _C
_NPAD = 10240
_RPT = _NPAD // _NS

_BLPAD = 3328
_GPW = _BLPAD // _NW


def _embed_support_body(type_f, length_f, lane_f, node_tab, type_tab,
                        length_tab, lane_tab, w, b, init_ref, sup_ref):
    tf = type_f[...]
    lf = length_f[...]
    nf = lane_f[...]
    lane_oh = (nf[:, None] == lax.broadcasted_iota(jnp.int32, (_N, 10), 1)
               ).astype(jnp.float32)
    type_oh = (tf[:, None] == lax.broadcasted_iota(jnp.int32, (_N, 20), 1)
               ).astype(jnp.float32)
    len_oh = (lf[:, None] == lax.broadcasted_iota(jnp.int32, (_N, 100), 1)
              ).astype(jnp.float32)
    lane_emb = jnp.dot(lane_oh, lane_tab[...], preferred_element_type=jnp.float32)
    type_emb = jnp.dot(type_oh, type_tab[...], preferred_element_type=jnp.float32)
    len_emb = jnp.dot(len_oh, length_tab[...], preferred_element_type=jnp.float32)
    feat = jnp.concatenate([lane_emb, type_emb, len_emb, node_tab[...]], axis=1)
    init_ref[...] = feat
    sup_ref[...] = jnp.dot(feat, w[...], preferred_element_type=jnp.float32) + b[...]


def _embed_support(type_f, length_f, lane_f, node_tab, type_tab, length_tab,
                   lane_tab, w, b2d):
    return pl.pallas_call(
        _embed_support_body,
        out_shape=(jax.ShapeDtypeStruct((_N, _HID), jnp.float32),
                   jax.ShapeDtypeStruct((_N, _HID), jnp.float32)),
    )(type_f, length_f, lane_f, node_tab, type_tab, length_tab, lane_tab, w, b2d)


def _spmm_body(sup_hbm, src_hbm, dst_hbm, val_hbm, zeros_hbm, out_hbm,
               acc_shared, src_v, dst_v, val_v, rows_v, sem):
    c = lax.axis_index("c")
    s = lax.axis_index("s")
    wid = c * _NS + s
    base = wid * _EPW
    pltpu.sync_copy(zeros_hbm, acc_shared.at[pl.ds(s * _RPT, _RPT)])
    plsc.subcore_barrier()

    def chunk(k, carry):
        off = base + k * _C
        pltpu.sync_copy(src_hbm.at[pl.ds(off, _C)], src_v)
        pltpu.sync_copy(dst_hbm.at[pl.ds(off, _C)], dst_v)
        pltpu.sync_copy(val_hbm.at[pl.ds(off, _C)], val_v)
        pltpu.async_copy(sup_hbm.at[src_v], rows_v, sem).wait()

        def scale(kk, carry2):
            valv = val_v[pl.ds(kk * 16, 16)]
            for e16 in range(16):
                vb = lax.gather(
                    valv, jnp.full((16, 1), e16, jnp.int32),
                    lax.GatherDimensionNumbers(
                        offset_dims=(), collapsed_slice_dims=(0,),
                        start_index_map=(0,)),
                    (1,), mode=lax.GatherScatterMode.PROMISE_IN_BOUNDS)
                e = kk * 16 + e16
                for f in range(_HID // 16):
                    rows_v[e, pl.ds(f * 16, 16)] = (
                        rows_v[e, pl.ds(f * 16, 16)] * vb)
            return carry2

        lax.fori_loop(0, _C // 16, scale, 0)
        pltpu.sync_copy(rows_v, acc_shared.at[dst_v], add=True)
        return carry

    lax.fori_loop(0, _NCHUNK, chunk, 0)
    plsc.subcore_barrier()
    pltpu.sync_copy(acc_shared.at[pl.ds(s * _RPT, _RPT)],
                    out_hbm.at[c, pl.ds(s * _RPT, _RPT)])


def _spmm(support, src, dst, val, zeros):
    mesh = plsc.VectorSubcoreMesh(core_axis_name="c", subcore_axis_name="s")
    return pl.kernel(
        _spmm_body,
        out_type=jax.ShapeDtypeStruct((_NC, _NPAD, _HID), jnp.float32),
        mesh=mesh,
        scratch_types=[
            pltpu.VMEM_SHARED((_NPAD, _HID), jnp.float32),
            pltpu.VMEM((_C,), jnp.int32),
            pltpu.VMEM((_C,), jnp.int32),
            pltpu.VMEM((_C,), jnp.float32),
            pltpu.VMEM((_C, _HID), jnp.float32),
            pltpu.SemaphoreType.DMA,
        ],
    )(support, src, dst, val, zeros)


def _relu_support_body(p, w, b, sup_ref):
    h = jnp.maximum(p[0, 0:_N, :] + p[1, 0:_N, :], 0.0)
    sup_ref[...] = jnp.dot(h, w[...], preferred_element_type=jnp.float32) + b[...]


def _relu_support(partial, w, b2d):
    return pl.pallas_call(
        _relu_support_body,
        out_shape=jax.ShapeDtypeStruct((_N, _HID), jnp.float32),
    )(partial, w, b2d)


def _relu_body(p, out_ref):
    out_ref[...] = jnp.maximum(p[0, 0:_N, :] + p[1, 0:_N, :], 0.0)


def _relu_combine(partial):
    return pl.pallas_call(
        _relu_body,
        out_shape=jax.ShapeDtypeStruct((_N, _HID), jnp.float32),
    )(partial)


def _gather_body(t0_hbm, t1_hbm, idx_hbm, out0_hbm, out1_hbm,
                 idx_v, rows_v, sem):
    c = lax.axis_index("c")
    s = lax.axis_index("s")
    wid = c * _NS + s
    base = wid * _GPW
    pltpu.sync_copy(idx_hbm.at[pl.ds(base, _GPW)], idx_v)
    pltpu.async_copy(t0_hbm.at[idx_v], rows_v, sem).wait()
    pltpu.sync_copy(rows_v, out0_hbm.at[pl.ds(base, _GPW)])
    pltpu.async_copy(t1_hbm.at[idx_v], rows_v, sem).wait()
    pltpu.sync_copy(rows_v, out1_hbm.at[pl.ds(base, _GPW)])


def _traj_gather(t0, t1, idx_pad):
    mesh = plsc.VectorSubcoreMesh(core_axis_name="c", subcore_axis_name="s")
    return pl.kernel(
        _gather_body,
        out_type=(jax.ShapeDtypeStruct((_BLPAD, _HID), jnp.float32),
                  jax.ShapeDtypeStruct((_BLPAD, _HID), jnp.float32)),
        mesh=mesh,
        scratch_types=[
            pltpu.VMEM((_GPW,), jnp.int32),
            pltpu.VMEM((_GPW, _HID), jnp.float32),
            pltpu.SemaphoreType.DMA,
        ],
    )(t0, t1, idx_pad)


def _gru_body(x, w_ih, w_hh, b_ih, b_hh, out_ref, gi_ref):
    gi_ref[...] = lax.dot_general(
        x[...], w_ih[...], (((1,), (1,)), ((), ())),
        preferred_element_type=jnp.float32) + b_ih[...]

    def step(l, h):
        gi = gi_ref[pl.ds(l * _B, _B), :]
        gh = lax.dot_general(h, w_hh[...], (((1,), (1,)), ((), ())),
                             preferred_element_type=jnp.float32) + b_hh[...]
        i_r = gi[:, 0:_HID]
        i_z = gi[:, _HID:2 * _HID]
        i_n = gi[:, 2 * _HID:3 * _HID]
        h_r = gh[:, 0:_HID]
        h_z = gh[:, _HID:2 * _HID]
        h_n = gh[:, 2 * _HID:3 * _HID]
        r = jax.nn.sigmoid(i_r + h_r)
        z = jax.nn.sigmoid(i_z + h_z)
        n = jnp.tanh(i_n + r * h_n)
        h_new = (1.0 - z) * n + z * h
        out_ref[pl.ds(l * _B, _B), :] = h_new
        return h_new

    lax.fori_loop(0, _L, step, jnp.zeros((_B, _HID), jnp.float32))


def _gru(x_flat, w_ih, w_hh, b_ih2d, b_hh2d):
    return pl.pallas_call(
        _gru_body,
        out_shape=jax.ShapeDtypeStruct((_BL, _HID), jnp.float32),
        scratch_shapes=[pltpu.VMEM((_BL, 3 * _HID), jnp.float32)],
    )(x_flat, w_ih, w_hh, b_ih2d, b_hh2d)


_NBLK = 512


def _lin_body(outs, gcn, w, b, o_ref):
    acc = lax.dot_general(outs[...], w[:, 0:_HID], (((1,), (1,)), ((), ())),
                          preferred_element_type=jnp.float32)
    acc = acc + lax.dot_general(gcn[...], w[:, _HID:2 * _HID],
                                (((1,), (1,)), ((), ())),
                                preferred_element_type=jnp.float32)
    o_ref[...] = acc + b[...]


def _final_linear(outs, gcn, lin_w, lin_b2d):
    nblocks = pl.cdiv(_N, _NBLK)
    return pl.pallas_call(
        _lin_body,
        grid=(nblocks,),
        in_specs=[
            pl.BlockSpec((_BL, _HID), lambda i: (0, 0)),
            pl.BlockSpec((_BL, _HID), lambda i: (0, 0)),
            pl.BlockSpec((_NBLK, 2 * _HID), lambda i: (i, 0)),
            pl.BlockSpec((1, _NBLK), lambda i: (0, i)),
        ],
        out_specs=pl.BlockSpec((_BL, _NBLK), lambda i: (0, i)),
        out_shape=jax.ShapeDtypeStruct((_BL, _N), jnp.float32),
    )(outs, gcn, lin_w, lin_b2d)


def kernel(input_bat, node_feature, type_feature, length_feature, lane_feature,
           adj_index, adj_values, node_tab, type_tab, length_tab, lane_tab,
           gcn_W, gcn_b, W_ih, W_hh, b_ih, b_hh, lin_W, lin_b):
    b2d = gcn_b.reshape(1, _HID)
    init_feat, support = _embed_support(
        type_feature, length_feature, lane_feature, node_tab, type_tab,
        length_tab, lane_tab, gcn_W, b2d)

    dst = adj_index[0]
    src = adj_index[1]
    zeros = jnp.zeros((_RPT, _HID), jnp.float32)

    part1 = _spmm(support, src, dst, adj_values, zeros)
    support2 = _relu_support(part1, gcn_W, b2d)
    part2 = _spmm(support2, src, dst, adj_values, zeros)
    node_out = _relu_combine(part2)

    idx_pad = jnp.concatenate(
        [input_bat.reshape(_BL), jnp.zeros((_BLPAD - _BL,), jnp.int32)])
    g_init, g_gcn = _traj_gather(init_feat, node_out, idx_pad)
    g_init = g_init[:_BL]
    g_gcn = g_gcn[:_BL]

    outs = _gru(g_init, W_ih, W_hh, b_ih.reshape(1, 3 * _HID),
                b_hh.reshape(1, 3 * _HID))
    pred = _final_linear(outs, g_gcn, lin_W, lin_b.reshape(1, _N))
    return pred.reshape(_L, _B, _N)


# speedup vs baseline: 8.1295x; 2.2653x over previous
import functools

import jax
import jax.numpy as jnp
from jax import lax
from jax.experimental import pallas as pl
from jax.experimental.pallas import tpu as pltpu
from jax.experimental.pallas import tpu_sc as plsc

_N = 10000
_E = 320000
_HID = 128
_B = 64
_L = 50
_BL = _B * _L

_NC = 2
_NS = 16
_NW = _NC * _NS
_EPW = _E // _NW
_C = 80
_NCHUNK = _EPW // _C
_NPAD = 10240
_RPT = _NPAD // _NS

_BLPAD = 3328
_GPW = _BLPAD // _NW


def _embed_support_body(type_f, length_f, lane_f, node_tab, type_tab,
                        length_tab, lane_tab, w, b, init_ref, sup_ref):
    tf = type_f[...]
    lf = length_f[...]
    nf = lane_f[...]
    lane_oh = (nf[:, None] == lax.broadcasted_iota(jnp.int32, (_N, 10), 1)
               ).astype(jnp.float32)
    type_oh = (tf[:, None] == lax.broadcasted_iota(jnp.int32, (_N, 20), 1)
               ).astype(jnp.float32)
    len_oh = (lf[:, None] == lax.broadcasted_iota(jnp.int32, (_N, 100), 1)
              ).astype(jnp.float32)
    lane_emb = jnp.dot(lane_oh, lane_tab[...], preferred_element_type=jnp.float32)
    type_emb = jnp.dot(type_oh, type_tab[...], preferred_element_type=jnp.float32)
    len_emb = jnp.dot(len_oh, length_tab[...], preferred_element_type=jnp.float32)
    feat = jnp.concatenate([lane_emb, type_emb, len_emb, node_tab[...]], axis=1)
    init_ref[...] = feat
    sup_ref[...] = jnp.dot(feat, w[...], preferred_element_type=jnp.float32) + b[...]


def _embed_support(type_f, length_f, lane_f, node_tab, type_tab, length_tab,
                   lane_tab, w, b2d):
    return pl.pallas_call(
        _embed_support_body,
        out_shape=(jax.ShapeDtypeStruct((_N, _HID), jnp.float32),
                   jax.ShapeDtypeStruct((_N, _HID), jnp.float32)),
    )(type_f, length_f, lane_f, node_tab, type_tab, length_tab, lane_tab, w, b2d)


def _spmm_body(sup_hbm, src_hbm, dst_hbm, val_hbm, out_hbm,
               acc_shared, src_all, val_all, rows_a, rows_b,
               dst_ca, dst_cb, sem_a, sem_b, sem_da, sem_db, sem_i):
    c = lax.axis_index("c")
    s = lax.axis_index("s")
    wid = c * _NS + s
    base = wid * _EPW
    pltpu.async_copy(src_hbm.at[pl.ds(base, _EPW)], src_all, sem_i).wait()
    pltpu.async_copy(val_hbm.at[pl.ds(base, _EPW)], val_all, sem_i).wait()

    def zrow(i, carry):
        for f in range(_HID // 16):
            rows_a[i, pl.ds(f * 16, 16)] = jnp.zeros((16,), jnp.float32)
        return carry

    lax.fori_loop(0, _C, zrow, 0)
    for j in range(_RPT // _C):
        pltpu.sync_copy(rows_a, acc_shared.at[pl.ds(s * _RPT + j * _C, _C)])
    plsc.subcore_barrier()

    def gather_rows(k, rows, sem):
        return pltpu.make_async_copy(
            sup_hbm.at[src_all.at[pl.ds(k * _C, _C)]], rows, sem)

    def fetch_dst(k, dst_c, sem):
        return pltpu.make_async_copy(
            dst_hbm.at[pl.ds(base + k * _C, _C)], dst_c, sem)

    def process(k, rows, dst_c, sem_d):
        def scale(kk, carry2):
            o = k * _C + kk * 16
            valv = val_all[pl.ds(o, 16)]
            for e16 in range(16):
                vb = lax.gather(
                    valv, jnp.full((16, 1), e16, jnp.int32),
                    lax.GatherDimensionNumbers(
                        offset_dims=(), collapsed_slice_dims=(0,),
                        start_index_map=(0,)),
                    (1,), mode=lax.GatherScatterMode.PROMISE_IN_BOUNDS)
                e = kk * 16 + e16
                for f in range(_HID // 16):
                    rows[e, pl.ds(f * 16, 16)] = rows[e, pl.ds(f * 16, 16)] * vb
            return carry2

        lax.fori_loop(0, _C // 16, scale, 0)
        fetch_dst(k, dst_c, sem_d).wait()
        pltpu.sync_copy(rows, acc_shared.at[dst_c], add=True)

    fetch_dst(0, dst_ca, sem_da).start()
    fetch_dst(1, dst_cb, sem_db).start()
    gather_rows(0, rows_a, sem_a).start()
    gather_rows(1, rows_b, sem_b).start()

    def pipe(i, carry):
        ka = 2 * i
        gather_rows(ka, rows_a, sem_a).wait()
        process(ka, rows_a, dst_ca, sem_da)

        @pl.when(i < (_NCHUNK - 1) // 2)
        def _():
            fetch_dst(ka + 2, dst_ca, sem_da).start()
            gather_rows(ka + 2, rows_a, sem_a).start()

        gather_rows(ka + 1, rows_b, sem_b).wait()
        process(ka + 1, rows_b, dst_cb, sem_db)

        @pl.when(i < (_NCHUNK - 1) // 2 - 1)
        def _():
            fetch_dst(ka + 3, dst_cb, sem_db).start()
            gather_rows(ka + 3, rows_b, sem_b).start()
        return carry

    lax.fori_loop(0, (_NCHUNK - 1) // 2, pipe, 0)
    gather_rows(_NCHUNK - 1, rows_a, sem_a).wait()
    process(_NCHUNK - 1, rows_a, dst_ca, sem_da)

    plsc.subcore_barrier()
    pltpu.sync_copy(acc_shared.at[pl.ds(s * _RPT, _RPT)],
                    out_hbm.at[c, pl.ds(s * _RPT, _RPT)])


def _spmm(support, src, dst, val):
    mesh = plsc.VectorSubcoreMesh(core_axis_name="c", subcore_axis_name="s")
    return pl.kernel(
        _spmm_body,
        out_type=jax.ShapeDtypeStruct((_NC, _NPAD, _HID), jnp.float32),
        mesh=mesh,
        compiler_params=pltpu.CompilerParams(use_tc_tiling_on_sc=False),
        scratch_types=[
            pltpu.VMEM_SHARED((_NPAD, _HID), jnp.float32),
            pltpu.VMEM((_EPW,), jnp.int32),
            pltpu.VMEM((_EPW,), jnp.float32),
            pltpu.VMEM((_C, _HID), jnp.float32),
            pltpu.VMEM((_C, _HID), jnp.float32),
            pltpu.VMEM((_C,), jnp.int32),
            pltpu.VMEM((_C,), jnp.int32),
            pltpu.SemaphoreType.DMA,
            pltpu.SemaphoreType.DMA,
            pltpu.SemaphoreType.DMA,
            pltpu.SemaphoreType.DMA,
            pltpu.SemaphoreType.DMA,
        ],
    )(support, src, dst, val)


def _relu_support_body(p, w, b, sup_ref):
    h = jnp.maximum(p[0, 0:_N, :] + p[1, 0:_N, :], 0.0)
    sup_ref[...] = jnp.dot(h, w[...], preferred_element_type=jnp.float32) + b[...]


def _relu_support(partial, w, b2d):
    return pl.pallas_call(
        _relu_support_body,
        out_shape=jax.ShapeDtypeStruct((_N, _HID), jnp.float32),
    )(partial, w, b2d)


def _relu_body(p, out_ref):
    out_ref[...] = jnp.maximum(p[0, 0:_N, :] + p[1, 0:_N, :], 0.0)


def _relu_combine(partial):
    return pl.pallas_call(
        _relu_body,
        out_shape=jax.ShapeDtypeStruct((_N, _HID), jnp.float32),
    )(partial)


def _gather_body(t0_hbm, t1_hbm, idx_hbm, out0_hbm, out1_hbm,
                 idx_v, rows_v, sem):
    c = lax.axis_index("c")
    s = lax.axis_index("s")
    wid = c * _NS + s
    base = wid * _GPW
    pltpu.sync_copy(idx_hbm.at[pl.ds(base, _GPW)], idx_v)
    pltpu.async_copy(t0_hbm.at[idx_v], rows_v, sem).wait()
    pltpu.sync_copy(rows_v, out0_hbm.at[pl.ds(base, _GPW)])
    pltpu.async_copy(t1_hbm.at[idx_v], rows_v, sem).wait()
    pltpu.sync_copy(rows_v, out1_hbm.at[pl.ds(base, _GPW)])


def _traj_gather(t0, t1, idx_pad):
    mesh = plsc.VectorSubcoreMesh(core_axis_name="c", subcore_axis_name="s")
    return pl.kernel(
        _gather_body,
        out_type=(jax.ShapeDtypeStruct((_BLPAD, _HID), jnp.float32),
                  jax.ShapeDtypeStruct((_BLPAD, _HID), jnp.float32)),
        mesh=mesh,
        scratch_types=[
            pltpu.VMEM((_GPW,), jnp.int32),
            pltpu.VMEM((_GPW, _HID), jnp.float32),
            pltpu.SemaphoreType.DMA,
        ],
    )(t0, t1, idx_pad)


def _gru_body(x, w_ih, w_hh, b_ih, b_hh, out_ref, gi_ref):
    gi_ref[...] = lax.dot_general(
        x[...], w_ih[...], (((1,), (1,)), ((), ())),
        preferred_element_type=jnp.float32) + b_ih[...]

    def step(l, h):
        gi = gi_ref[pl.ds(l * _B, _B), :]
        gh = lax.dot_general(h, w_hh[...], (((1,), (1,)), ((), ())),
                             preferred_element_type=jnp.float32) + b_hh[...]
        i_r = gi[:, 0:_HID]
        i_z = gi[:, _HID:2 * _HID]
        i_n = gi[:, 2 * _HID:3 * _HID]
        h_r = gh[:, 0:_HID]
        h_z = gh[:, _HID:2 * _HID]
        h_n = gh[:, 2 * _HID:3 * _HID]
        r = jax.nn.sigmoid(i_r + h_r)
        z = jax.nn.sigmoid(i_z + h_z)
        n = jnp.tanh(i_n + r * h_n)
        h_new = (1.0 - z) * n + z * h
        out_ref[pl.ds(l * _B, _B), :] = h_new
        return h_new

    lax.fori_loop(0, _L, step, jnp.zeros((_B, _HID), jnp.float32))


def _gru(x_flat, w_ih, w_hh, b_ih2d, b_hh2d):
    return pl.pallas_call(
        _gru_body,
        out_shape=jax.ShapeDtypeStruct((_BL, _HID), jnp.float32),
        scratch_shapes=[pltpu.VMEM((_BL, 3 * _HID), jnp.float32)],
    )(x_flat, w_ih, w_hh, b_ih2d, b_hh2d)


_NBLK = 512


def _lin_body(outs, gcn, w, b, o_ref):
    acc = lax.dot_general(outs[...], w[:, 0:_HID], (((1,), (1,)), ((), ())),
                          preferred_element_type=jnp.float32)
    acc = acc + lax.dot_general(gcn[...], w[:, _HID:2 * _HID],
                                (((1,), (1,)), ((), ())),
                                preferred_element_type=jnp.float32)
    o_ref[...] = acc + b[...]


def _final_linear(outs, gcn, lin_w, lin_b2d):
    nblocks = pl.cdiv(_N, _NBLK)
    return pl.pallas_call(
        _lin_body,
        grid=(nblocks,),
        in_specs=[
            pl.BlockSpec((_BL, _HID), lambda i: (0, 0)),
            pl.BlockSpec((_BL, _HID), lambda i: (0, 0)),
            pl.BlockSpec((_NBLK, 2 * _HID), lambda i: (i, 0)),
            pl.BlockSpec((1, _NBLK), lambda i: (0, i)),
        ],
        out_specs=pl.BlockSpec((_BL, _NBLK), lambda i: (0, i)),
        out_shape=jax.ShapeDtypeStruct((_BL, _N), jnp.float32),
    )(outs, gcn, lin_w, lin_b2d)


def kernel(input_bat, node_feature, type_feature, length_feature, lane_feature,
           adj_index, adj_values, node_tab, type_tab, length_tab, lane_tab,
           gcn_W, gcn_b, W_ih, W_hh, b_ih, b_hh, lin_W, lin_b):
    b2d = gcn_b.reshape(1, _HID)
    init_feat, support = _embed_support(
        type_feature, length_feature, lane_feature, node_tab, type_tab,
        length_tab, lane_tab, gcn_W, b2d)

    dst = adj_index[0]
    src = adj_index[1]

    part1 = _spmm(support, src, dst, adj_values)
    support2 = _relu_support(part1, gcn_W, b2d)
    part2 = _spmm(support2, src, dst, adj_values)
    node_out = _relu_combine(part2)

    idx_pad = jnp.concatenate(
        [input_bat.reshape(_BL), jnp.zeros((_BLPAD - _BL,), jnp.int32)])
    g_init, g_gcn = _traj_gather(init_feat, node_out, idx_pad)
    g_init = g_init[:_BL]
    g_gcn = g_gcn[:_BL]

    outs = _gru(g_init, W_ih, W_hh, b_ih.reshape(1, 3 * _HID),
                b_hh.reshape(1, 3 * _HID))
    pred = _final_linear(outs, g_gcn, lin_W, lin_b.reshape(1, _N))
    return pred.reshape(_L, _B, _N)
